# baseline (speedup 1.0000x reference)
import functools

import jax
import jax.numpy as jnp
from jax.experimental import pallas as pl
from jax.experimental.pallas import tpu as pltpu

_C = 1.0
_MARGIN = 0.2
_KNN = 5
_TEMP = 0.1
_NA = 32
_B = 1024
_EPS = 1e-3


def _acosh(x):
    return jnp.log(x + jnp.sqrt((x + 1.0) * (x - 1.0)))


def _loss_kernel(batch_ref, labc_ref, labr_ref, gpos_ref, gneg_ref, out_ref,
                 b_scr, x2_scr):
    B, NA = _B, _NA
    INF = jnp.inf

    x = batch_ref[:]
    lab_col = labc_ref[:]
    lab_row = labr_ref[:]

    norm = jnp.clip(jnp.sqrt(jnp.sum(x * x, axis=1, keepdims=True)), 1e-15, None)
    maxnorm = (1.0 - _EPS) / jnp.sqrt(_C)
    b = jnp.where(norm > maxnorm, x / norm * maxnorm, x)
    b_scr[:, :] = b

    x2c = jnp.sum(b * b, axis=1, keepdims=True)
    x2_scr[:, :] = x2c
    x2r = x2c.reshape(1, B)
    G = jax.lax.dot_general(b, b, (((1,), (1,)), ((), ())),
                            preferred_element_type=jnp.float32)
    sq = jnp.clip(x2c + x2r - 2.0 * G, 0.0, None)
    den = jnp.clip((1.0 - x2c) * (1.0 - x2r), 1e-15, None)
    arg = jnp.clip(1.0 + 2.0 * sq / den, 1.0 + 1e-7, None)

    iota_r = jax.lax.broadcasted_iota(jnp.int32, (B, B), 1)
    iota1 = jax.lax.broadcasted_iota(jnp.int32, (1, B), 1)
    iota_na = jax.lax.broadcasted_iota(jnp.int32, (NA, B), 1)

    work = arg
    nbr_idx = []
    for t in range(6):
        idx = jnp.argmin(work, axis=1).reshape(B, 1).astype(jnp.int32)
        nbr_idx.append(idx)
        if t < 5:
            work = jnp.where(iota_r == idx, INF, work)
    sel = (iota_r == nbr_idx[1]) | (iota_r == nbr_idx[2]) | \
          (iota_r == nbr_idx[3]) | (iota_r == nbr_idx[4]) | \
          (iota_r == nbr_idx[5])
    match = lab_row == lab_col
    ones_col = jnp.ones((B, 1), jnp.float32)
    cnt_f = jax.lax.dot_general((sel & match).astype(jnp.float32), ones_col,
                                (((1,), (0,)), ((), ())),
                                preferred_element_type=jnp.float32)
    cnt = cnt_f.astype(jnp.int32)

    cntr = cnt.reshape(1, B)
    iota_c0 = jax.lax.broadcasted_iota(jnp.int32, (B, B), 0)
    beats = (cnt > cntr) | ((cnt == cntr) & (iota_c0 < iota_r))
    ones_row = jnp.ones((1, B), jnp.float32)
    rank_f = jax.lax.dot_general(ones_row, beats.astype(jnp.float32),
                                 (((1,), (0,)), ((), ())),
                                 preferred_element_type=jnp.float32)
    rank = rank_f.astype(jnp.int32)

    a_idx_l = []
    ba_rows = []
    x2a_rows = []
    lab_a_rows = []
    for p in range(NA):
        ai = jnp.min(jnp.where(rank == p, iota1, B), axis=1, keepdims=True)
        i = ai[0, 0]
        a_idx_l.append(ai)
        ba_rows.append(b_scr[pl.ds(i, 1), :])
        x2a_rows.append(x2_scr[pl.ds(i, 1), :])
        lab_a_rows.append(labc_ref[pl.ds(i, 1), :])
    a_idx = jnp.concatenate(a_idx_l, axis=0)
    ba = jnp.concatenate(ba_rows, axis=0)
    x2a = jnp.concatenate(x2a_rows, axis=0)
    a_lab = jnp.concatenate(lab_a_rows, axis=0)

    Ga = jax.lax.dot_general(ba, b, (((1,), (1,)), ((), ())),
                             preferred_element_type=jnp.float32)
    sq_a = jnp.clip(x2a + x2r - 2.0 * Ga, 0.0, None)
    den_a = jnp.clip((1.0 - x2a) * (1.0 - x2r), 1e-15, None)
    arg_a = jnp.clip(1.0 + 2.0 * sq_a / den_a, 1.0 + 1e-7, None)

    Da = _acosh(arg_a)
    same = lab_row == a_lab
    not_self = iota_na != a_idx
    pos_logits = jnp.where(same & not_self, -Da / _TEMP, -INF)
    neg_logits = jnp.where(~same, Da / _TEMP, -INF)

    P = gpos_ref[:] + pos_logits
    mp = jnp.max(P, axis=1, keepdims=True)
    pos_idx = jnp.min(jnp.where(P == mp, iota_na, B), axis=1, keepdims=True)
    N = gneg_ref[:] + neg_logits
    mn = jnp.max(N, axis=1, keepdims=True)
    neg_idx = jnp.min(jnp.where(N == mn, iota_na, B), axis=1, keepdims=True)

    bp_rows = []
    bn_rows = []
    for p in range(NA):
        bp_rows.append(b_scr[pl.ds(pos_idx[p, 0], 1), :])
        bn_rows.append(b_scr[pl.ds(neg_idx[p, 0], 1), :])
    bp = jnp.concatenate(bp_rows, axis=0)
    bn = jnp.concatenate(bn_rows, axis=0)

    def dist(u, v):
        s = jnp.sum((u - v) ** 2, axis=1, keepdims=True)
        u2 = jnp.sum(u * u, axis=1, keepdims=True)
        v2 = jnp.sum(v * v, axis=1, keepdims=True)
        dn_ = jnp.clip((1.0 - u2) * (1.0 - v2), 1e-15, None)
        a = jnp.clip(1.0 + 2.0 * s / dn_, 1.0 + 1e-7, None)
        return _acosh(a)

    dp = dist(ba, bp)
    dn = dist(ba, bn)
    loss = jnp.sum(jnp.maximum(dp - dn + _MARGIN, 0.0),
                   axis=0, keepdims=True) / NA
    out_ref[:, :] = loss


@functools.partial(jax.jit, static_argnames=())
def kernel(batch, labels):
    labels = labels.astype(jnp.int32)
    kp, kn = jax.random.split(jax.random.key(42))
    g_pos = jax.random.gumbel(kp, (_NA, _B), jnp.float32)
    g_neg = jax.random.gumbel(kn, (_NA, _B), jnp.float32)
    out = pl.pallas_call(
        _loss_kernel,
        out_shape=jax.ShapeDtypeStruct((1, 1), jnp.float32),
        scratch_shapes=[
            pltpu.VMEM((_B, 128), jnp.float32),
            pltpu.VMEM((_B, 1), jnp.float32),
        ],
    )(batch, labels.reshape(_B, 1), labels.reshape(1, _B), g_pos, g_neg)
    return out[0, 0]


# speedup vs baseline: 1.0244x; 1.0244x over previous
import functools

import jax
import jax.numpy as jnp
from jax.experimental import pallas as pl
from jax.experimental.pallas import tpu as pltpu

_C = 1.0
_MARGIN = 0.2
_KNN = 5
_TEMP = 0.1
_NA = 32
_B = 1024
_EPS = 1e-3


def _acosh(x):
    return jnp.log(x + jnp.sqrt((x + 1.0) * (x - 1.0)))


def _loss_kernel(batch_ref, labc_ref, labr_ref, gpos_ref, gneg_ref, out_ref,
                 b_scr, x2_scr):
    B, NA = _B, _NA
    INF = jnp.inf

    x = batch_ref[:]
    lab_col = labc_ref[:]
    lab_row = labr_ref[:]

    norm = jnp.clip(jnp.sqrt(jnp.sum(x * x, axis=1, keepdims=True)), 1e-15, None)
    maxnorm = (1.0 - _EPS) / jnp.sqrt(_C)
    b = jnp.where(norm > maxnorm, x / norm * maxnorm, x)
    b_scr[:, :] = b

    x2c = jnp.sum(b * b, axis=1, keepdims=True)
    x2_scr[:, :] = x2c
    x2r = x2c.reshape(1, B)
    G = jax.lax.dot_general(b, b, (((1,), (1,)), ((), ())),
                            preferred_element_type=jnp.float32)
    sq = jnp.clip(x2c + x2r - 2.0 * G, 0.0, None)
    den = jnp.clip((1.0 - x2c) * (1.0 - x2r), 1e-15, None)
    arg = jnp.clip(1.0 + 2.0 * sq / den, 1.0 + 1e-7, None)

    iota_r = jax.lax.broadcasted_iota(jnp.int32, (B, B), 1)
    iota1 = jax.lax.broadcasted_iota(jnp.int32, (1, B), 1)
    iota_na = jax.lax.broadcasted_iota(jnp.int32, (NA, B), 1)

    work = arg
    sel = None
    for t in range(6):
        idx = jnp.argmin(work, axis=1).reshape(B, 1).astype(jnp.int32)
        hit = iota_r == idx
        if t > 0:
            sel = hit if sel is None else sel | hit
        if t < 5:
            work = jnp.where(hit, INF, work)
    match = lab_row == lab_col
    cnt = jnp.sum((sel & match).astype(jnp.int32), axis=1, keepdims=True)

    cntr = cnt.reshape(1, B)
    iota_c0 = jax.lax.broadcasted_iota(jnp.int32, (B, B), 0)
    beats = (cnt > cntr) | ((cnt == cntr) & (iota_c0 < iota_r))
    rank = jnp.sum(beats.astype(jnp.int32), axis=0, keepdims=True)

    a_idx_l = []
    ba_rows = []
    x2a_rows = []
    lab_a_rows = []
    for p in range(NA):
        ai = jnp.min(jnp.where(rank == p, iota1, B), axis=1, keepdims=True)
        i = ai[0, 0]
        a_idx_l.append(ai)
        ba_rows.append(b_scr[pl.ds(i, 1), :])
        x2a_rows.append(x2_scr[pl.ds(i, 1), :])
        lab_a_rows.append(labc_ref[pl.ds(i, 1), :])
    a_idx = jnp.concatenate(a_idx_l, axis=0)
    ba = jnp.concatenate(ba_rows, axis=0)
    x2a = jnp.concatenate(x2a_rows, axis=0)
    a_lab = jnp.concatenate(lab_a_rows, axis=0)

    Ga = jax.lax.dot_general(ba, b, (((1,), (1,)), ((), ())),
                             preferred_element_type=jnp.float32)
    sq_a = jnp.clip(x2a + x2r - 2.0 * Ga, 0.0, None)
    den_a = jnp.clip((1.0 - x2a) * (1.0 - x2r), 1e-15, None)
    arg_a = jnp.clip(1.0 + 2.0 * sq_a / den_a, 1.0 + 1e-7, None)

    Da = _acosh(arg_a)
    same = lab_row == a_lab
    not_self = iota_na != a_idx
    pos_logits = jnp.where(same & not_self, -Da / _TEMP, -INF)
    neg_logits = jnp.where(~same, Da / _TEMP, -INF)

    P = gpos_ref[:] + pos_logits
    mp = jnp.max(P, axis=1, keepdims=True)
    pos_idx = jnp.min(jnp.where(P == mp, iota_na, B), axis=1, keepdims=True)
    N = gneg_ref[:] + neg_logits
    mn = jnp.max(N, axis=1, keepdims=True)
    neg_idx = jnp.min(jnp.where(N == mn, iota_na, B), axis=1, keepdims=True)

    bp_rows = []
    bn_rows = []
    for p in range(NA):
        bp_rows.append(b_scr[pl.ds(pos_idx[p, 0], 1), :])
        bn_rows.append(b_scr[pl.ds(neg_idx[p, 0], 1), :])
    bp = jnp.concatenate(bp_rows, axis=0)
    bn = jnp.concatenate(bn_rows, axis=0)

    def dist(u, v):
        s = jnp.sum((u - v) ** 2, axis=1, keepdims=True)
        u2 = jnp.sum(u * u, axis=1, keepdims=True)
        v2 = jnp.sum(v * v, axis=1, keepdims=True)
        dn_ = jnp.clip((1.0 - u2) * (1.0 - v2), 1e-15, None)
        a = jnp.clip(1.0 + 2.0 * s / dn_, 1.0 + 1e-7, None)
        return _acosh(a)

    dp = dist(ba, bp)
    dn = dist(ba, bn)
    loss = jnp.sum(jnp.maximum(dp - dn + _MARGIN, 0.0),
                   axis=0, keepdims=True) / NA
    out_ref[:, :] = loss


@functools.partial(jax.jit, static_argnames=())
def kernel(batch, labels):
    labels = labels.astype(jnp.int32)
    kp, kn = jax.random.split(jax.random.key(42))
    g_pos = jax.random.gumbel(kp, (_NA, _B), jnp.float32)
    g_neg = jax.random.gumbel(kn, (_NA, _B), jnp.float32)
    out = pl.pallas_call(
        _loss_kernel,
        out_shape=jax.ShapeDtypeStruct((1, 1), jnp.float32),
        scratch_shapes=[
            pltpu.VMEM((_B, 128), jnp.float32),
            pltpu.VMEM((_B, 1), jnp.float32),
        ],
    )(batch, labels.reshape(_B, 1), labels.reshape(1, _B), g_pos, g_neg)
    return out[0, 0]
